# Initial kernel scaffold; baseline (speedup 1.0000x reference)
import jax
import jax.numpy as jnp
from jax.experimental import pallas as pl


def kernel(token_ids, table):
    raise NotImplementedError("write your pallas kernel here")


# speedup vs baseline: 4.5741x; 4.5741x over previous
import functools

import jax
import jax.numpy as jnp
from jax import lax
from jax.experimental import pallas as pl
from jax.experimental.pallas import tpu as pltpu
from jax.experimental.pallas import tpu_sc as plsc

NUM_CORES = 2
NUM_SUBCORES = 16
NW = NUM_CORES * NUM_SUBCORES
CH = 128
GRP = 5


def _emb_body(n_ch, d, idx_hbm, table_hbm, out_hbm, idx_v, rows_v, gsem):
    wid = lax.axis_index("s") * NUM_CORES + lax.axis_index("c")
    per_w = n_ch * CH
    base = wid * per_w
    pltpu.sync_copy(idx_hbm.at[wid], idx_v)

    n_grp = n_ch // GRP

    @pl.loop(0, n_grp)
    def _group(g):
        descs = [
            pltpu.async_copy(
                table_hbm.at[idx_v.at[g * GRP + i]],
                rows_v.at[pl.ds(i * CH, CH)],
                gsem,
            )
            for i in range(GRP)
        ]
        for dsc in descs:
            dsc.wait()
        pltpu.sync_copy(rows_v, out_hbm.at[pl.ds(base + g * (GRP * CH), GRP * CH)])


def kernel(token_ids, table):
    b, s = token_ids.shape
    v, d = table.shape
    n = b * s
    assert n % (NW * CH) == 0
    n_ch = n // (NW * CH)
    assert n_ch % GRP == 0

    idx = token_ids.reshape(NW, n_ch, CH).astype(jnp.int32)

    mesh = plsc.VectorSubcoreMesh(core_axis_name="c", subcore_axis_name="s")
    emb = functools.partial(
        pl.kernel,
        out_type=jax.ShapeDtypeStruct((n, d), jnp.float32),
        mesh=mesh,
        scratch_types=[
            pltpu.VMEM((n_ch, CH), jnp.int32),
            pltpu.VMEM((GRP * CH, d), jnp.float32),
            pltpu.SemaphoreType.DMA,
        ],
        compiler_params=pltpu.CompilerParams(use_tc_tiling_on_sc=False),
    )(functools.partial(_emb_body, n_ch, d))

    out = emb(idx, table)
    return out.reshape(b, s, d)


# speedup vs baseline: 4.6205x; 1.0101x over previous
import functools

import jax
import jax.numpy as jnp
from jax import lax
from jax.experimental import pallas as pl
from jax.experimental.pallas import tpu as pltpu
from jax.experimental.pallas import tpu_sc as plsc

NUM_CORES = 2
NUM_SUBCORES = 16
NW = NUM_CORES * NUM_SUBCORES
CH = 128
GRP = 5


def _emb_body(n_ch, d, idx_hbm, table_hbm, out_hbm,
              idx_v, rows_v, gsem0, gsem1, osem0, osem1):
    wid = lax.axis_index("s") * NUM_CORES + lax.axis_index("c")
    rows_per_grp = GRP * CH
    base = wid * n_ch * CH
    n_grp = n_ch // GRP
    gsems = (gsem0, gsem1)
    osems = (osem0, osem1)

    pltpu.sync_copy(idx_hbm.at[wid], idx_v)

    def gather_descs(gg, p):
        return [
            pltpu.make_async_copy(
                table_hbm.at[idx_v.at[gg * GRP + i]],
                rows_v.at[p].at[pl.ds(i * CH, CH)],
                gsems[p],
            )
            for i in range(GRP)
        ]

    def wb_desc(gg, p):
        return pltpu.make_async_copy(
            rows_v.at[p],
            out_hbm.at[pl.ds(base + gg * rows_per_grp, rows_per_grp)],
            osems[p],
        )

    for dsc in gather_descs(0, 0):
        dsc.start()

    @pl.loop(0, n_grp, step=2)
    def _group(g):
        for p in range(2):
            gg = g + p
            for dsc in gather_descs(gg, p):
                dsc.wait()
            wb_desc(gg, p).start()

            @pl.when(gg >= 1)
            def _wait_prev_wb():
                wb_desc(gg - 1, 1 - p).wait()

            @pl.when(gg + 1 < n_grp)
            def _fire_next():
                for dsc in gather_descs(gg + 1, 1 - p):
                    dsc.start()

    wb_desc(n_grp - 1, (n_grp - 1) % 2).wait()


def kernel(token_ids, table):
    b, s = token_ids.shape
    v, d = table.shape
    n = b * s
    assert n % (NW * CH) == 0
    n_ch = n // (NW * CH)
    n_grp = n_ch // GRP
    assert n_ch % GRP == 0 and n_grp % 2 == 0

    idx = token_ids.reshape(NW, n_ch, CH).astype(jnp.int32)

    mesh = plsc.VectorSubcoreMesh(core_axis_name="c", subcore_axis_name="s")
    emb = functools.partial(
        pl.kernel,
        out_type=jax.ShapeDtypeStruct((n, d), jnp.float32),
        mesh=mesh,
        scratch_types=[
            pltpu.VMEM((n_ch, CH), jnp.int32),
            pltpu.VMEM((2, GRP * CH, d), jnp.float32),
            pltpu.SemaphoreType.DMA,
            pltpu.SemaphoreType.DMA,
            pltpu.SemaphoreType.DMA,
            pltpu.SemaphoreType.DMA,
        ],
        compiler_params=pltpu.CompilerParams(use_tc_tiling_on_sc=False),
    )(functools.partial(_emb_body, n_ch, d))

    out = emb(idx, table)
    return out.reshape(b, s, d)
